# baseline (speedup 1.0000x reference)
import functools

import jax
import jax.numpy as jnp
from jax import lax
from jax.experimental import pallas as pl
from jax.experimental.pallas import tpu as pltpu
from jax.experimental.pallas import tpu_sc as plsc

BATCH = 16384
DIM = 32
TABLE = 1000000
L = 16
NC, NS = 2, 16
NW = NC * NS
BPW = BATCH // NW
CHUNK = 128
NCH = BPW // CHUNK

UT = (TABLE + 127) // 128
NOCT = DIM // 8
HOCT = NOCT // 2
OCT_STRIDE = UT * 1024
OCT_JUMP = OCT_STRIDE - 7 * 128
TPC = 16
TC_TPC = 64

_mesh = plsc.VectorSubcoreMesh(core_axis_name="c", subcore_axis_name="s")


def _retile_tc_body(uref, iref, uout, iout):
    for t in range(TC_TPC):
        s = pl.ds(t * 128, 128)
        uout[0, t] = uref[:, s]
        iout[0, t] = iref[:, s]


_retile_tc = pl.pallas_call(
    _retile_tc_body,
    grid=(HOCT, (UT + TC_TPC - 1) // TC_TPC),
    in_specs=[
        pl.BlockSpec((8, TC_TPC * 128), lambda o, c: (o, c)),
        pl.BlockSpec((8, TC_TPC * 128), lambda o, c: (o, c)),
    ],
    out_specs=[
        pl.BlockSpec((1, TC_TPC, 8, 128), lambda o, c: (o, c, 0, 0)),
        pl.BlockSpec((1, TC_TPC, 8, 128), lambda o, c: (o, c, 0, 0)),
    ],
    out_shape=[
        jax.ShapeDtypeStruct((HOCT, UT, 8, 128), jnp.float32),
        jax.ShapeDtypeStruct((HOCT, UT, 8, 128), jnp.float32),
    ],
)


@functools.partial(
    pl.kernel,
    out_type=(jax.ShapeDtypeStruct((HOCT, UT, 8, 128), jnp.float32),
              jax.ShapeDtypeStruct((HOCT, UT, 8, 128), jnp.float32)),
    mesh=_mesh,
    compiler_params=pltpu.CompilerParams(needs_layout_passes=False,
                                         use_tc_tiling_on_sc=True),
    scratch_types=[
        pltpu.VMEM((TPC, 8, 128), jnp.float32),
        pltpu.VMEM((TPC, 8, 128), jnp.float32),
        pltpu.SemaphoreType.DMA,
        pltpu.SemaphoreType.DMA,
    ],
)
def _retile_sc(uembt, iembt, tail_u, tail_i, uout, iout,
               buf_a, buf_b, sem_a, sem_b):
    wid = lax.axis_index("s") * NC + lax.axis_index("c")
    nfull = (UT - 1) // TPC
    nper = (nfull + NW - 1) // NW

    def do_chunk(src, dst, c, buf, sem):
        ut0 = c * TPC
        cs = []
        for t in range(TPC):
            cs.append(pltpu.async_copy(
                src.at[:, pl.ds((ut0 + t) * 128, 128)], buf.at[t], sem))
        for cpy in cs:
            cpy.wait()
        pltpu.async_copy(buf, dst.at[0, pl.ds(ut0, TPC)], sem).wait()

    def body(k, carry):
        c = jnp.minimum(wid * nper + k, nfull - 1)
        for o in range(HOCT):
            sl_u = uembt.at[pl.ds((HOCT + o) * 8, 8), :]
            sl_i = iembt.at[pl.ds((HOCT + o) * 8, 8), :]
            do_chunk(sl_u, uout.at[pl.ds(o, 1)], c, buf_a, sem_a)
            do_chunk(sl_i, iout.at[pl.ds(o, 1)], c, buf_b, sem_b)
        return carry

    lax.fori_loop(0, nper, body, 0)

    @pl.when(wid == 0)
    def _():
        for src, tail, dst, buf, sem in ((uembt, tail_u, uout, buf_a, sem_a),
                                         (iembt, tail_i, iout, buf_b, sem_b)):
            for o in range(HOCT):
                for ut in range(nfull * TPC, UT - 1):
                    pltpu.sync_copy(
                        src.at[pl.ds((HOCT + o) * 8, 8), pl.ds(ut * 128, 128)],
                        buf.at[0])
                    pltpu.async_copy(buf.at[0], dst.at[o, ut], sem).wait()
                pltpu.sync_copy(tail.at[o], buf.at[0])
                pltpu.async_copy(buf.at[0], dst.at[o, UT - 1], sem).wait()


@functools.partial(
    pl.kernel,
    out_type=jax.ShapeDtypeStruct((BATCH,), jnp.float32),
    mesh=_mesh,
    compiler_params=pltpu.CompilerParams(needs_layout_passes=False,
                                         use_tc_tiling_on_sc=False),
    scratch_types=[
        pltpu.VMEM((NCH, CHUNK), jnp.int32),
        pltpu.VMEM((NCH, CHUNK), jnp.int32),
        pltpu.VMEM((NCH, CHUNK), jnp.int32),
        pltpu.VMEM((NCH, CHUNK), jnp.int32),
        pltpu.VMEM((BPW,), jnp.float32),
        pltpu.VMEM((BPW,), jnp.float32),
        pltpu.VMEM((BPW,), jnp.float32),
        pltpu.VMEM((BPW,), jnp.float32),
        pltpu.VMEM((BPW,), jnp.float32),
        pltpu.VMEM((BPW,), jnp.float32),
        pltpu.VMEM((BPW,), jnp.float32),
        pltpu.SemaphoreType.DMA,
        pltpu.SemaphoreType.DMA,
        pltpu.SemaphoreType.DMA,
    ],
)
def _mf_bpr(uid, iid, uflat_lo, iflat_lo, uflat_hi, iflat_hi, ubias, ibias, out,
            uidx, iidx, ufx, ifx, ua, ia, ub_c, ib_c, ub, ib, out_v,
            sema, semb, bsem):
    wid = lax.axis_index("s") * NC + lax.axis_index("c")
    base = wid * BPW

    for j in range(NCH):
        pltpu.sync_copy(uid.at[pl.ds(base + j * CHUNK, CHUNK)], uidx.at[j])
        pltpu.sync_copy(iid.at[pl.ds(base + j * CHUNK, CHUNK)], iidx.at[j])
    for j in range(NCH):
        for k in range(CHUNK // L):
            s = pl.ds(k * L, L)
            uidx[j, s] = uidx[j, s] - 1
            iidx[j, s] = iidx[j, s] - 1

    bias_copies = []
    for j in range(NCH):
        rs = pl.ds(j * CHUNK, CHUNK)
        bias_copies.append(pltpu.async_copy(ubias.at[uidx.at[j]], ub.at[rs], bsem))
        bias_copies.append(pltpu.async_copy(ibias.at[iidx.at[j]], ib.at[rs], bsem))

    zeros = jnp.zeros((L,), jnp.float32)
    for t in range(BPW // L):
        out_v[pl.ds(t * L, L)] = zeros

    def seed():
        for j in range(NCH):
            for k in range(CHUNK // L):
                s = pl.ds(k * L, L)
                u = uidx[j, s]
                i = iidx[j, s]
                ufx[j, s] = lax.shift_left(lax.shift_right_logical(u, 7), 10) + (u & 127)
                ifx[j, s] = lax.shift_left(lax.shift_right_logical(i, 7), 10) + (i & 127)

    def advance(amt):
        for j in range(NCH):
            for k in range(CHUNK // L):
                s = pl.ds(k * L, L)
                ufx[j, s] = ufx[j, s] + amt
                ifx[j, s] = ifx[j, s] + amt

    def accumulate(src_u, src_i):
        for t in range(BPW // L):
            s = pl.ds(t * L, L)
            out_v[s] = out_v[s] + src_u[s] * src_i[s]

    def half(uflat, iflat):
        def fire(dst_u, dst_i, sem):
            for j in range(NCH):
                rs = pl.ds(j * CHUNK, CHUNK)
                pltpu.async_copy(uflat.at[ufx.at[j]], dst_u.at[rs], sem)
                pltpu.async_copy(iflat.at[ifx.at[j]], dst_i.at[rs], sem)

        def drain(dst_u, dst_i, sem):
            for j in range(NCH):
                rs = pl.ds(j * CHUNK, CHUNK)
                pltpu.make_async_copy(uflat.at[ufx.at[j]], dst_u.at[rs], sem).wait()
                pltpu.make_async_copy(iflat.at[ifx.at[j]], dst_i.at[rs], sem).wait()

        seed()
        fire(ua, ia, sema)

        def body(g, carry):
            drain(ua, ia, sema)
            advance(jnp.int32(128))
            fire(ub_c, ib_c, semb)
            accumulate(ua, ia)
            drain(ub_c, ib_c, semb)

            @pl.when(g < (8 * HOCT // 2 - 1))
            def _():
                advance(jnp.where((g & 3) == 3, jnp.int32(OCT_JUMP),
                                  jnp.int32(128)))
                fire(ua, ia, sema)

            accumulate(ub_c, ib_c)
            return carry

        lax.fori_loop(0, 8 * HOCT // 2, body, 0)

    half(uflat_lo, iflat_lo)
    half(uflat_hi, iflat_hi)

    for c in bias_copies:
        c.wait()
    for t in range(BPW // L):
        s = pl.ds(t * L, L)
        out_v[s] = out_v[s] + ub[s] + ib[s]

    pltpu.sync_copy(out_v, out.at[pl.ds(base, BPW)])


def kernel(user_id, item_id, user_embedding, item_embedding, user_bias, item_bias):
    uembt = user_embedding.T
    iembt = item_embedding.T
    tail0 = (UT - 1) * 128
    tw = TABLE - tail0
    tail_u = jnp.pad(uembt[HOCT * 8:, tail0:].reshape(HOCT, 8, tw),
                     ((0, 0), (0, 0), (0, 128 - tw)))
    tail_i = jnp.pad(iembt[HOCT * 8:, tail0:].reshape(HOCT, 8, tw),
                     ((0, 0), (0, 0), (0, 128 - tw)))
    u_hi, i_hi = _retile_sc(uembt, iembt, tail_u, tail_i)
    u_lo, i_lo = _retile_tc(uembt, iembt)
    return _mf_bpr(user_id, item_id,
                   u_lo.reshape(-1), i_lo.reshape(-1),
                   u_hi.reshape(-1), i_hi.reshape(-1),
                   user_bias.reshape(-1), item_bias.reshape(-1))


# speedup vs baseline: 1.0501x; 1.0501x over previous
import functools

import jax
import jax.numpy as jnp
from jax import lax
from jax.experimental import pallas as pl
from jax.experimental.pallas import tpu as pltpu
from jax.experimental.pallas import tpu_sc as plsc

BATCH = 16384
DIM = 32
TABLE = 1000000
L = 16
NC, NS = 2, 16
NW = NC * NS
BPW = BATCH // NW
CHUNK = 128
NCH = BPW // CHUNK

UT = (TABLE + 127) // 128
NOCT = DIM // 8
HOCT = NOCT // 2
OCT_STRIDE = UT * 1024
OCT_JUMP = OCT_STRIDE - 7 * 128
TPC = 16
TC_TPC = 64

_mesh = plsc.VectorSubcoreMesh(core_axis_name="c", subcore_axis_name="s")


def _retile_tc_body(uref, iref, uout, iout):
    for t in range(TC_TPC):
        s = pl.ds(t * 128, 128)
        uout[0, t] = uref[:, s]
        iout[0, t] = iref[:, s]


_retile_tc = pl.pallas_call(
    _retile_tc_body,
    grid=(HOCT, (UT + TC_TPC - 1) // TC_TPC),
    in_specs=[
        pl.BlockSpec((8, TC_TPC * 128), lambda o, c: (o, c)),
        pl.BlockSpec((8, TC_TPC * 128), lambda o, c: (o, c)),
    ],
    out_specs=[
        pl.BlockSpec((1, TC_TPC, 8, 128), lambda o, c: (o, c, 0, 0)),
        pl.BlockSpec((1, TC_TPC, 8, 128), lambda o, c: (o, c, 0, 0)),
    ],
    out_shape=[
        jax.ShapeDtypeStruct((HOCT, UT, 8, 128), jnp.float32),
        jax.ShapeDtypeStruct((HOCT, UT, 8, 128), jnp.float32),
    ],
)


@functools.partial(
    pl.kernel,
    out_type=(jax.ShapeDtypeStruct((HOCT, UT, 8, 128), jnp.float32),
              jax.ShapeDtypeStruct((HOCT, UT, 8, 128), jnp.float32)),
    mesh=_mesh,
    compiler_params=pltpu.CompilerParams(needs_layout_passes=False,
                                         use_tc_tiling_on_sc=True),
    scratch_types=[
        pltpu.VMEM((TPC, 8, 128), jnp.float32),
        pltpu.VMEM((TPC, 8, 128), jnp.float32),
        pltpu.SemaphoreType.DMA,
        pltpu.SemaphoreType.DMA,
    ],
)
def _retile_sc(uembt, iembt, tail_u, tail_i, uout, iout,
               buf_a, buf_b, sem_a, sem_b):
    wid = lax.axis_index("s") * NC + lax.axis_index("c")
    nfull = (UT - 1) // TPC
    nper = (nfull + NW - 1) // NW

    def do_chunk(src, dst, c, buf, sem):
        ut0 = c * TPC
        cs = []
        for t in range(TPC):
            cs.append(pltpu.async_copy(
                src.at[:, pl.ds((ut0 + t) * 128, 128)], buf.at[t], sem))
        for cpy in cs:
            cpy.wait()
        pltpu.async_copy(buf, dst.at[0, pl.ds(ut0, TPC)], sem).wait()

    def body(k, carry):
        c = jnp.minimum(wid * nper + k, nfull - 1)
        for o in range(HOCT):
            sl_u = uembt.at[pl.ds((HOCT + o) * 8, 8), :]
            sl_i = iembt.at[pl.ds((HOCT + o) * 8, 8), :]
            do_chunk(sl_u, uout.at[pl.ds(o, 1)], c, buf_a, sem_a)
            do_chunk(sl_i, iout.at[pl.ds(o, 1)], c, buf_b, sem_b)
        return carry

    lax.fori_loop(0, nper, body, 0)

    @pl.when(wid == 0)
    def _():
        for src, tail, dst, buf, sem in ((uembt, tail_u, uout, buf_a, sem_a),
                                         (iembt, tail_i, iout, buf_b, sem_b)):
            for o in range(HOCT):
                for ut in range(nfull * TPC, UT - 1):
                    pltpu.sync_copy(
                        src.at[pl.ds((HOCT + o) * 8, 8), pl.ds(ut * 128, 128)],
                        buf.at[0])
                    pltpu.async_copy(buf.at[0], dst.at[o, ut], sem).wait()
                pltpu.sync_copy(tail.at[o], buf.at[0])
                pltpu.async_copy(buf.at[0], dst.at[o, UT - 1], sem).wait()


@functools.partial(
    pl.kernel,
    out_type=jax.ShapeDtypeStruct((BATCH,), jnp.float32),
    mesh=_mesh,
    compiler_params=pltpu.CompilerParams(needs_layout_passes=False,
                                         use_tc_tiling_on_sc=False),
    scratch_types=[
        pltpu.VMEM((NCH, CHUNK), jnp.int32),
        pltpu.VMEM((NCH, CHUNK), jnp.int32),
        pltpu.VMEM((NCH, CHUNK), jnp.int32),
        pltpu.VMEM((NCH, CHUNK), jnp.int32),
        pltpu.VMEM((NCH, CHUNK), jnp.int32),
        pltpu.VMEM((NCH, CHUNK), jnp.int32),
        pltpu.VMEM((BPW,), jnp.float32),
        pltpu.VMEM((BPW,), jnp.float32),
        pltpu.VMEM((BPW,), jnp.float32),
        pltpu.VMEM((BPW,), jnp.float32),
        pltpu.VMEM((BPW,), jnp.float32),
        pltpu.VMEM((BPW,), jnp.float32),
        pltpu.VMEM((BPW,), jnp.float32),
        pltpu.SemaphoreType.DMA,
        pltpu.SemaphoreType.DMA,
        pltpu.SemaphoreType.DMA,
    ],
)
def _mf_bpr(uid, iid, uflat_lo, iflat_lo, uflat_hi, iflat_hi, ubias, ibias, out,
            uidx, iidx, ufxa, ifxa, ufxb, ifxb, ua, ia, ub_c, ib_c, ub, ib,
            out_v, sema, semb, bsem):
    wid = lax.axis_index("s") * NC + lax.axis_index("c")
    base = wid * BPW

    for j in range(NCH):
        pltpu.sync_copy(uid.at[pl.ds(base + j * CHUNK, CHUNK)], uidx.at[j])
        pltpu.sync_copy(iid.at[pl.ds(base + j * CHUNK, CHUNK)], iidx.at[j])
    for j in range(NCH):
        for k in range(CHUNK // L):
            s = pl.ds(k * L, L)
            uidx[j, s] = uidx[j, s] - 1
            iidx[j, s] = iidx[j, s] - 1

    bias_copies = []
    for j in range(NCH):
        rs = pl.ds(j * CHUNK, CHUNK)
        bias_copies.append(pltpu.async_copy(ubias.at[uidx.at[j]], ub.at[rs], bsem))
        bias_copies.append(pltpu.async_copy(ibias.at[iidx.at[j]], ib.at[rs], bsem))

    zeros = jnp.zeros((L,), jnp.float32)
    for t in range(BPW // L):
        out_v[pl.ds(t * L, L)] = zeros

    def seed(ufx, ifx, off):
        for j in range(NCH):
            for k in range(CHUNK // L):
                s = pl.ds(k * L, L)
                u = uidx[j, s]
                i = iidx[j, s]
                ufx[j, s] = (lax.shift_left(lax.shift_right_logical(u, 7), 10)
                             + (u & 127) + off)
                ifx[j, s] = (lax.shift_left(lax.shift_right_logical(i, 7), 10)
                             + (i & 127) + off)

    def advance(ufx, ifx, amt):
        for j in range(NCH):
            for k in range(CHUNK // L):
                s = pl.ds(k * L, L)
                ufx[j, s] = ufx[j, s] + amt
                ifx[j, s] = ifx[j, s] + amt

    def accumulate(src_u, src_i):
        for t in range(BPW // L):
            s = pl.ds(t * L, L)
            out_v[s] = out_v[s] + src_u[s] * src_i[s]

    STEP2_JUMP = OCT_STRIDE - 6 * 128

    def half(uflat, iflat):
        def fire(ufx, ifx, dst_u, dst_i, sem):
            for j in range(NCH):
                rs = pl.ds(j * CHUNK, CHUNK)
                pltpu.async_copy(uflat.at[ufx.at[j]], dst_u.at[rs], sem)
                pltpu.async_copy(iflat.at[ifx.at[j]], dst_i.at[rs], sem)

        def drain(ufx, ifx, dst_u, dst_i, sem):
            for j in range(NCH):
                rs = pl.ds(j * CHUNK, CHUNK)
                pltpu.make_async_copy(uflat.at[ufx.at[j]], dst_u.at[rs], sem).wait()
                pltpu.make_async_copy(iflat.at[ifx.at[j]], dst_i.at[rs], sem).wait()

        seed(ufxa, ifxa, jnp.int32(0))
        fire(ufxa, ifxa, ua, ia, sema)
        seed(ufxb, ifxb, jnp.int32(128))
        fire(ufxb, ifxb, ub_c, ib_c, semb)

        def body(g, carry):
            step2 = jnp.where((g & 3) == 3, jnp.int32(STEP2_JUMP),
                              jnp.int32(256))
            drain(ufxa, ifxa, ua, ia, sema)
            accumulate(ua, ia)

            @pl.when(g < (8 * HOCT // 2 - 1))
            def _():
                advance(ufxa, ifxa, step2)
                fire(ufxa, ifxa, ua, ia, sema)

            drain(ufxb, ifxb, ub_c, ib_c, semb)
            accumulate(ub_c, ib_c)

            @pl.when(g < (8 * HOCT // 2 - 1))
            def _():
                advance(ufxb, ifxb, step2)
                fire(ufxb, ifxb, ub_c, ib_c, semb)

            return carry

        lax.fori_loop(0, 8 * HOCT // 2, body, 0)

    half(uflat_lo, iflat_lo)
    half(uflat_hi, iflat_hi)

    for c in bias_copies:
        c.wait()
    for t in range(BPW // L):
        s = pl.ds(t * L, L)
        out_v[s] = out_v[s] + ub[s] + ib[s]

    pltpu.sync_copy(out_v, out.at[pl.ds(base, BPW)])


def kernel(user_id, item_id, user_embedding, item_embedding, user_bias, item_bias):
    uembt = user_embedding.T
    iembt = item_embedding.T
    tail0 = (UT - 1) * 128
    tw = TABLE - tail0
    tail_u = jnp.pad(uembt[HOCT * 8:, tail0:].reshape(HOCT, 8, tw),
                     ((0, 0), (0, 0), (0, 128 - tw)))
    tail_i = jnp.pad(iembt[HOCT * 8:, tail0:].reshape(HOCT, 8, tw),
                     ((0, 0), (0, 0), (0, 128 - tw)))
    u_hi, i_hi = _retile_sc(uembt, iembt, tail_u, tail_i)
    u_lo, i_lo = _retile_tc(uembt, iembt)
    return _mf_bpr(user_id, item_id,
                   u_lo.reshape(-1), i_lo.reshape(-1),
                   u_hi.reshape(-1), i_hi.reshape(-1),
                   user_bias.reshape(-1), item_bias.reshape(-1))
